# baseline (speedup 1.0000x reference)
import jax
import jax.numpy as jnp
from jax.experimental import pallas as pl
from jax.experimental.pallas import tpu as pltpu

_R = 128
_C = 8192
_SUB = 128


def _scan_kernel(x_ref, o_ref, carry_ref):
    ci = pl.program_id(0)
    ri = pl.program_id(1)
    rbase = ri * _R

    @pl.when(ci == 0)
    def _():
        carry_ref[pl.ds(rbase, _R), :] = jnp.zeros((_R, _SUB), jnp.float32)

    x = x_ref[...]
    T = (jax.lax.broadcasted_iota(jnp.int32, (_SUB, _SUB), 0)
         < jax.lax.broadcasted_iota(jnp.int32, (_SUB, _SUB), 1)
         ).astype(jnp.float32)
    ones = jnp.ones((_SUB, _SUB), jnp.float32)
    B = jnp.concatenate([T, ones], axis=1)
    carry = carry_ref[pl.ds(rbase, _R), :]
    for k in range(_C // _SUB):
        xs = x[:, k * _SUB:(k + 1) * _SUB]
        y = jnp.dot(xs, B, preferred_element_type=jnp.float32)
        o_ref[:, k * _SUB:(k + 1) * _SUB] = y[:, :_SUB] + carry
        carry = carry + y[:, _SUB:]
    carry_ref[pl.ds(rbase, _R), :] = carry


@jax.jit
def kernel(x):
    m, n = x.shape
    grid = (n // _C, m // _R)
    return pl.pallas_call(
        _scan_kernel,
        grid=grid,
        in_specs=[pl.BlockSpec((_R, _C), lambda j, i: (i, j))],
        out_specs=pl.BlockSpec((_R, _C), lambda j, i: (i, j)),
        out_shape=jax.ShapeDtypeStruct((m, n), x.dtype),
        scratch_shapes=[pltpu.VMEM((m, _SUB), jnp.float32)],
        compiler_params=pltpu.CompilerParams(
            dimension_semantics=("arbitrary", "arbitrary")),
    )(x)


# speedup vs baseline: 1.0219x; 1.0219x over previous
import jax
import jax.numpy as jnp
from jax.experimental import pallas as pl
from jax.experimental.pallas import tpu as pltpu

_R = 256
_C = 8192
_SUB = 128


def _scan_kernel(x_ref, o_ref, carry_ref):
    ci = pl.program_id(0)
    ri = pl.program_id(1)
    rbase = ri * _R

    @pl.when(ci == 0)
    def _():
        carry_ref[pl.ds(rbase, _R), :] = jnp.zeros((_R, _SUB), jnp.float32)

    x = x_ref[...]
    T = (jax.lax.broadcasted_iota(jnp.int32, (_SUB, _SUB), 0)
         <= jax.lax.broadcasted_iota(jnp.int32, (_SUB, _SUB), 1)
         ).astype(jnp.float32)
    carry = carry_ref[pl.ds(rbase, _R), :]
    for k in range(_C // _SUB):
        xs = x[:, k * _SUB:(k + 1) * _SUB]
        incl = jnp.dot(xs, T, preferred_element_type=jnp.float32)
        o_ref[:, k * _SUB:(k + 1) * _SUB] = incl - xs + carry
        carry = carry + incl[:, _SUB - 1:_SUB]
    carry_ref[pl.ds(rbase, _R), :] = carry


@jax.jit
def kernel(x):
    m, n = x.shape
    grid = (n // _C, m // _R)
    return pl.pallas_call(
        _scan_kernel,
        grid=grid,
        in_specs=[pl.BlockSpec((_R, _C), lambda j, i: (i, j))],
        out_specs=pl.BlockSpec((_R, _C), lambda j, i: (i, j)),
        out_shape=jax.ShapeDtypeStruct((m, n), x.dtype),
        scratch_shapes=[pltpu.VMEM((m, _SUB), jnp.float32)],
        compiler_params=pltpu.CompilerParams(
            dimension_semantics=("arbitrary", "arbitrary")),
    )(x)


# speedup vs baseline: 1.0313x; 1.0092x over previous
import jax
import jax.numpy as jnp
from jax.experimental import pallas as pl
from jax.experimental.pallas import tpu as pltpu

_R = 256
_SUB = 128


def _scan_kernel(x_ref, o_ref):
    n = x_ref.shape[1]
    x = x_ref[...]
    T = (jax.lax.broadcasted_iota(jnp.int32, (_SUB, _SUB), 0)
         < jax.lax.broadcasted_iota(jnp.int32, (_SUB, _SUB), 1)
         ).astype(jnp.float32)
    ones = jnp.ones((_SUB, _SUB), jnp.float32)
    B = jnp.concatenate([T, ones], axis=1)
    carry = jnp.zeros((_R, _SUB), jnp.float32)
    for k in range(n // _SUB):
        xs = x[:, k * _SUB:(k + 1) * _SUB]
        y = jnp.dot(xs, B, preferred_element_type=jnp.float32)
        o_ref[:, k * _SUB:(k + 1) * _SUB] = y[:, :_SUB] + carry
        carry = carry + y[:, _SUB:]


@jax.jit
def kernel(x):
    m, n = x.shape
    return pl.pallas_call(
        _scan_kernel,
        grid=(m // _R,),
        in_specs=[pl.BlockSpec((_R, n), lambda i: (i, 0))],
        out_specs=pl.BlockSpec((_R, n), lambda i: (i, 0)),
        out_shape=jax.ShapeDtypeStruct((m, n), x.dtype),
        compiler_params=pltpu.CompilerParams(
            dimension_semantics=("arbitrary",)),
    )(x)


# speedup vs baseline: 1.0324x; 1.0011x over previous
import jax
import jax.numpy as jnp
from jax.experimental import pallas as pl
from jax.experimental.pallas import tpu as pltpu

_R = 256
_SUB = 128


def _scan_kernel(x_ref, o_ref):
    n = x_ref.shape[1]
    x = x_ref[...]
    T = (jax.lax.broadcasted_iota(jnp.int32, (_SUB, _SUB), 0)
         < jax.lax.broadcasted_iota(jnp.int32, (_SUB, _SUB), 1)
         ).astype(jnp.float32)
    ones = jnp.ones((_SUB, _SUB), jnp.float32)
    B = jnp.concatenate([T, ones], axis=1)
    carry = jnp.zeros((_R, _SUB), jnp.float32)
    for k in range(n // _SUB):
        xs = x[:, k * _SUB:(k + 1) * _SUB]
        y = jnp.dot(xs, B, preferred_element_type=jnp.float32)
        o_ref[:, k * _SUB:(k + 1) * _SUB] = y[:, :_SUB] + carry
        carry = carry + y[:, _SUB:]


@jax.jit
def kernel(x):
    m, n = x.shape
    return pl.pallas_call(
        _scan_kernel,
        grid=(m // _R,),
        in_specs=[pl.BlockSpec((_R, n), lambda i: (i, 0))],
        out_specs=pl.BlockSpec((_R, n), lambda i: (i, 0)),
        out_shape=jax.ShapeDtypeStruct((m, n), x.dtype),
        compiler_params=pltpu.CompilerParams(
            dimension_semantics=("parallel",)),
    )(x)
